# baseline (speedup 1.0000x reference)
import functools

import jax
import jax.numpy as jnp
from jax.experimental import pallas as pl
from jax.experimental.pallas import tpu as pltpu


def _se_kernel(x_ref, w1_ref, w2_ref, o_ref, *, inv_hw):
    x = x_ref[...]
    pooled = jnp.sum(x, axis=1, dtype=jnp.float32) * inv_hw
    h = jax.lax.dot_general(pooled, w1_ref[...],
                            (((1,), (1,)), ((), ())),
                            preferred_element_type=jnp.float32)
    h = jnp.maximum(h, 0.2 * h)
    y = jnp.tanh(jax.lax.dot_general(h, w2_ref[...],
                                     (((1,), (1,)), ((), ())),
                                     preferred_element_type=jnp.float32))
    o_ref[...] = x * y[:, None, :].astype(o_ref.dtype)


def _se_chunk(x, w1, w2):
    B, C, H, W = x.shape
    HW = H * W
    Cr = w1.shape[0]

    bytes_per_image = C * HW * x.dtype.itemsize
    tb_cap = max(1, (6 << 20) // bytes_per_image)
    tb = 1
    for cand in range(min(B, tb_cap), 0, -1):
        if B % cand == 0:
            tb = cand
            break

    x_t = x.reshape(B, C, HW).transpose(0, 2, 1)
    block = (tb, HW, C)
    block_bytes = tb * bytes_per_image
    vmem_limit = int(min(5 * block_bytes + (4 << 20), 56 << 20))

    out = pl.pallas_call(
        functools.partial(_se_kernel, inv_hw=1.0 / HW),
        out_shape=jax.ShapeDtypeStruct((B, HW, C), x.dtype),
        grid=(B // tb,),
        in_specs=[
            pl.BlockSpec(block, lambda b: (b, 0, 0)),
            pl.BlockSpec((Cr, C), lambda b: (0, 0)),
            pl.BlockSpec((C, Cr), lambda b: (0, 0)),
        ],
        out_specs=pl.BlockSpec(block, lambda b: (b, 0, 0)),
        compiler_params=pltpu.CompilerParams(
            dimension_semantics=("parallel",),
            vmem_limit_bytes=vmem_limit,
        ),
        cost_estimate=pl.CostEstimate(
            flops=2 * B * C * HW + 4 * B * C * Cr,
            transcendentals=B * C,
            bytes_accessed=2 * B * C * HW * x.dtype.itemsize,
        ),
    )(x_t, w1, w2)
    return out.transpose(0, 2, 1).reshape(B, C, H, W)


def kernel(x, w1, w2):
    B = x.shape[0]
    n_chunks = 4
    while n_chunks > 1 and B % n_chunks != 0:
        n_chunks //= 2
    cb = B // n_chunks
    outs = [_se_chunk(x[i * cb:(i + 1) * cb], w1, w2) for i in range(n_chunks)]
    return jnp.concatenate(outs, axis=0) if len(outs) > 1 else outs[0]


# speedup vs baseline: 1.0260x; 1.0260x over previous
import functools

import jax
import jax.numpy as jnp
from jax.experimental import pallas as pl
from jax.experimental.pallas import tpu as pltpu


def _se_kernel(x_ref, w1_ref, w2_ref, o_ref, *, inv_hw):
    x = x_ref[...]
    pooled = jnp.sum(x, axis=1, dtype=jnp.float32) * inv_hw
    h = jax.lax.dot_general(pooled, w1_ref[...],
                            (((1,), (1,)), ((), ())),
                            preferred_element_type=jnp.float32)
    h = jnp.maximum(h, 0.2 * h)
    y = jnp.tanh(jax.lax.dot_general(h, w2_ref[...],
                                     (((1,), (1,)), ((), ())),
                                     preferred_element_type=jnp.float32))
    o_ref[...] = x * y[:, None, :].astype(o_ref.dtype)


def _se_chunk(x, w1, w2):
    B, C, H, W = x.shape
    HW = H * W
    Cr = w1.shape[0]

    bytes_per_image = C * HW * x.dtype.itemsize
    tb_cap = max(1, (6 << 20) // bytes_per_image)
    tb = 1
    for cand in range(min(B, tb_cap), 0, -1):
        if B % cand == 0:
            tb = cand
            break

    x_t = x.reshape(B, C, HW).transpose(0, 2, 1)
    block = (tb, HW, C)
    block_bytes = tb * bytes_per_image
    vmem_limit = int(min(5 * block_bytes + (4 << 20), 56 << 20))

    out = pl.pallas_call(
        functools.partial(_se_kernel, inv_hw=1.0 / HW),
        out_shape=jax.ShapeDtypeStruct((B, HW, C), x.dtype),
        grid=(B // tb,),
        in_specs=[
            pl.BlockSpec(block, lambda b: (b, 0, 0)),
            pl.BlockSpec((Cr, C), lambda b: (0, 0)),
            pl.BlockSpec((C, Cr), lambda b: (0, 0)),
        ],
        out_specs=pl.BlockSpec(block, lambda b: (b, 0, 0)),
        compiler_params=pltpu.CompilerParams(
            dimension_semantics=("parallel",),
            vmem_limit_bytes=vmem_limit,
        ),
        cost_estimate=pl.CostEstimate(
            flops=2 * B * C * HW + 4 * B * C * Cr,
            transcendentals=B * C,
            bytes_accessed=2 * B * C * HW * x.dtype.itemsize,
        ),
    )(x_t, w1, w2)
    return out.transpose(0, 2, 1).reshape(B, C, H, W)


def kernel(x, w1, w2):
    B = x.shape[0]
    n_chunks = 2
    while n_chunks > 1 and B % n_chunks != 0:
        n_chunks //= 2
    cb = B // n_chunks
    outs = [_se_chunk(x[i * cb:(i + 1) * cb], w1, w2) for i in range(n_chunks)]
    return jnp.concatenate(outs, axis=0) if len(outs) > 1 else outs[0]


# speedup vs baseline: 1.6039x; 1.5633x over previous
import functools

import jax
import jax.numpy as jnp
from jax.experimental import pallas as pl
from jax.experimental.pallas import tpu as pltpu


def _se_kernel(x_ref, w1_ref, w2_ref, o_ref, *, inv_hw):
    x = x_ref[...]
    pooled = jnp.sum(x, axis=1, dtype=jnp.float32) * inv_hw
    h = jax.lax.dot_general(pooled, w1_ref[...],
                            (((1,), (1,)), ((), ())),
                            preferred_element_type=jnp.float32)
    h = jnp.maximum(h, 0.2 * h)
    y = jnp.tanh(jax.lax.dot_general(h, w2_ref[...],
                                     (((1,), (1,)), ((), ())),
                                     preferred_element_type=jnp.float32))
    o_ref[...] = x * y[:, None, :].astype(o_ref.dtype)


def _se_chunk(x, w1, w2):
    B, C, H, W = x.shape
    HW = H * W
    Cr = w1.shape[0]

    bytes_per_image = C * HW * x.dtype.itemsize
    tb_cap = max(1, (8 << 20) // bytes_per_image)
    tb = 1
    for cand in range(min(B, tb_cap), 0, -1):
        if B % cand == 0:
            tb = cand
            break

    x_t = x.reshape(B, C, HW).transpose(0, 2, 1)
    block = (tb, HW, C)
    block_bytes = tb * bytes_per_image
    vmem_limit = int(min(5 * block_bytes + (4 << 20), 56 << 20))

    out = pl.pallas_call(
        functools.partial(_se_kernel, inv_hw=1.0 / HW),
        out_shape=jax.ShapeDtypeStruct((B, HW, C), x.dtype),
        grid=(B // tb,),
        in_specs=[
            pl.BlockSpec(block, lambda b: (b, 0, 0)),
            pl.BlockSpec((Cr, C), lambda b: (0, 0)),
            pl.BlockSpec((C, Cr), lambda b: (0, 0)),
        ],
        out_specs=pl.BlockSpec(block, lambda b: (b, 0, 0)),
        compiler_params=pltpu.CompilerParams(
            dimension_semantics=("parallel",),
            vmem_limit_bytes=vmem_limit,
        ),
        cost_estimate=pl.CostEstimate(
            flops=2 * B * C * HW + 4 * B * C * Cr,
            transcendentals=B * C,
            bytes_accessed=2 * B * C * HW * x.dtype.itemsize,
        ),
    )(x_t, w1, w2)
    return out.transpose(0, 2, 1).reshape(B, C, H, W)


def kernel(x, w1, w2):
    B = x.shape[0]
    return _se_chunk(x, w1, w2)


# speedup vs baseline: 1.6178x; 1.0087x over previous
import functools

import jax
import jax.numpy as jnp
from jax.experimental import pallas as pl
from jax.experimental.pallas import tpu as pltpu


def _se_kernel(x_ref, w1_ref, w2_ref, o_ref, *, inv_hw):
    x = x_ref[...]
    pooled = jnp.sum(x, axis=1, dtype=jnp.float32) * inv_hw
    h = jax.lax.dot_general(pooled, w1_ref[...],
                            (((1,), (1,)), ((), ())),
                            preferred_element_type=jnp.float32)
    h = jnp.maximum(h, 0.2 * h)
    y = jnp.tanh(jax.lax.dot_general(h, w2_ref[...],
                                     (((1,), (1,)), ((), ())),
                                     preferred_element_type=jnp.float32))
    o_ref[...] = x * y[:, None, :].astype(o_ref.dtype)


def _se_chunk(x, w1, w2):
    B, C, H, W = x.shape
    HW = H * W
    Cr = w1.shape[0]

    bytes_per_image = C * HW * x.dtype.itemsize
    tb_cap = max(1, (13 << 20) // bytes_per_image)
    tb = 1
    for cand in range(min(B, tb_cap), 0, -1):
        if B % cand == 0:
            tb = cand
            break

    x_t = x.reshape(B, C, HW).transpose(0, 2, 1)
    block = (tb, HW, C)
    block_bytes = tb * bytes_per_image
    vmem_limit = int(min(5 * block_bytes + (4 << 20), 56 << 20))

    out = pl.pallas_call(
        functools.partial(_se_kernel, inv_hw=1.0 / HW),
        out_shape=jax.ShapeDtypeStruct((B, HW, C), x.dtype),
        grid=(B // tb,),
        in_specs=[
            pl.BlockSpec(block, lambda b: (b, 0, 0)),
            pl.BlockSpec((Cr, C), lambda b: (0, 0)),
            pl.BlockSpec((C, Cr), lambda b: (0, 0)),
        ],
        out_specs=pl.BlockSpec(block, lambda b: (b, 0, 0)),
        compiler_params=pltpu.CompilerParams(
            dimension_semantics=("parallel",),
            vmem_limit_bytes=vmem_limit,
        ),
        cost_estimate=pl.CostEstimate(
            flops=2 * B * C * HW + 4 * B * C * Cr,
            transcendentals=B * C,
            bytes_accessed=2 * B * C * HW * x.dtype.itemsize,
        ),
    )(x_t, w1, w2)
    return out.transpose(0, 2, 1).reshape(B, C, H, W)


def kernel(x, w1, w2):
    B = x.shape[0]
    return _se_chunk(x, w1, w2)
